# Initial kernel scaffold; baseline (speedup 1.0000x reference)
import functools

import jax
import jax.numpy as jnp
from jax import lax
from jax.experimental import pallas as pl
from jax.experimental.pallas import tpu as pltpu
from jax.experimental.pallas import tpu_sc as plsc

_VOCAB = 1000000
_EMBD = 32
_B = 16384 * 50

_NC = 2
_NS = 16
_NW = _NC * _NS
_BPW = _B // _NW
_C = 128
_NCHUNK = _BPW // _C

_mesh = plsc.VectorSubcoreMesh(core_axis_name="c", subcore_axis_name="s")


@functools.partial(
    pl.kernel,
    mesh=_mesh,
    out_type=jax.ShapeDtypeStruct((_B, _EMBD), jnp.float32),
    scratch_types=[
        pltpu.VMEM((_BPW,), jnp.int32),
        pltpu.VMEM((_C, _EMBD), jnp.float32),
        pltpu.SemaphoreType.DMA,
    ],
)
def _gather_kernel(idx_hbm, table_hbm, out_hbm, idx_v, rows_v, sem):
    wid = lax.axis_index("s") * _NC + lax.axis_index("c")
    base = wid * _BPW
    pltpu.sync_copy(idx_hbm.at[pl.ds(base, _BPW)], idx_v)

    def body(j, carry):
        off = j * _C
        pltpu.async_copy(
            table_hbm.at[idx_v.at[pl.ds(off, _C)]], rows_v, sem
        ).wait()
        pltpu.sync_copy(rows_v, out_hbm.at[pl.ds(base + off, _C)])
        return carry

    lax.fori_loop(0, _NCHUNK, body, 0)


def kernel(inputs, context_weight):
    idx = inputs.reshape(-1).astype(jnp.int32)
    out = _gather_kernel(idx, context_weight)
    return out.reshape(inputs.shape[0], inputs.shape[1], _EMBD)


# speedup vs baseline: 1.0223x; 1.0223x over previous
import functools

import jax
import jax.numpy as jnp
from jax import lax
from jax.experimental import pallas as pl
from jax.experimental.pallas import tpu as pltpu
from jax.experimental.pallas import tpu_sc as plsc

_VOCAB = 1000000
_EMBD = 32
_B = 16384 * 50

_NC = 2
_NS = 16
_NW = _NC * _NS
_BPW = _B // _NW
_C = 128
_NCHUNK = _BPW // _C

_mesh = plsc.VectorSubcoreMesh(core_axis_name="c", subcore_axis_name="s")


@functools.partial(
    pl.kernel,
    mesh=_mesh,
    out_type=jax.ShapeDtypeStruct((_B, _EMBD), jnp.float32),
    scratch_types=[
        pltpu.VMEM((_BPW,), jnp.int32),
        pltpu.VMEM((_C, _EMBD), jnp.float32),
        pltpu.SemaphoreType.DMA,
    ],
    compiler_params=pltpu.CompilerParams(use_tc_tiling_on_sc=False),
)
def _gather_kernel(idx_hbm, table_hbm, out_hbm, idx_v, rows_v, sem):
    wid = lax.axis_index("s") * _NC + lax.axis_index("c")
    base = wid * _BPW
    pltpu.sync_copy(idx_hbm.at[pl.ds(base, _BPW)], idx_v)

    def body(j, carry):
        off = j * _C
        pltpu.async_copy(
            table_hbm.at[idx_v.at[pl.ds(off, _C)]], rows_v, sem
        ).wait()
        pltpu.sync_copy(rows_v, out_hbm.at[pl.ds(base + off, _C)])
        return carry

    lax.fori_loop(0, _NCHUNK, body, 0)


def kernel(inputs, context_weight):
    idx = inputs.reshape(-1).astype(jnp.int32)
    out = _gather_kernel(idx, context_weight)
    return out.reshape(inputs.shape[0], inputs.shape[1], _EMBD)


# speedup vs baseline: 1.1114x; 1.0871x over previous
import functools

import jax
import jax.numpy as jnp
from jax import lax
from jax.experimental import pallas as pl
from jax.experimental.pallas import tpu as pltpu
from jax.experimental.pallas import tpu_sc as plsc

_EMBD = 32
_B = 16384 * 50

_NC = 2
_NS = 16
_NW = _NC * _NS
_BPW = _B // _NW
_C = 128
_N = _BPW // _C
_NBUF = 8
_LOOK = 4

_mesh = plsc.VectorSubcoreMesh(core_axis_name="c", subcore_axis_name="s")


@functools.partial(
    pl.kernel,
    mesh=_mesh,
    out_type=jax.ShapeDtypeStruct((_B, _EMBD), jnp.float32),
    scratch_types=(
        [pltpu.VMEM((_BPW,), jnp.int32)]
        + [pltpu.VMEM((_C, _EMBD), jnp.float32) for _ in range(_NBUF)]
        + [pltpu.SemaphoreType.DMA for _ in range(2 * _NBUF)]
    ),
    compiler_params=pltpu.CompilerParams(use_tc_tiling_on_sc=False),
)
def _gather_kernel(idx_hbm, table_hbm, out_hbm, idx_v, *bufs_and_sems):
    bufs = bufs_and_sems[:_NBUF]
    gsem = bufs_and_sems[_NBUF : 2 * _NBUF]
    wsem = bufs_and_sems[2 * _NBUF :]

    wid = lax.axis_index("s") * _NC + lax.axis_index("c")
    base = wid * _BPW
    pltpu.sync_copy(idx_hbm.at[pl.ds(base, _BPW)], idx_v)

    def start_gather(j, slot):
        pltpu.async_copy(
            table_hbm.at[idx_v.at[pl.ds(j * _C, _C)]], bufs[slot], gsem[slot]
        )

    def wait_gather(slot):
        pltpu.make_async_copy(
            table_hbm.at[idx_v.at[pl.ds(0, _C)]], bufs[slot], gsem[slot]
        ).wait()

    def start_write(j, slot):
        pltpu.async_copy(bufs[slot], out_hbm.at[pl.ds(base + j * _C, _C)], wsem[slot])

    def wait_write(slot):
        pltpu.make_async_copy(
            bufs[slot], out_hbm.at[pl.ds(0, _C)], wsem[slot]
        ).wait()

    for j in range(_LOOK):
        start_gather(j, j)

    for j in range(_LOOK):
        start_gather(j + _LOOK, j + _LOOK)
        wait_gather(j)
        start_write(j, j)

    @pl.loop(_LOOK, _N - _LOOK, step=_NBUF)
    def _steady(g):
        for b in range(_NBUF):
            j = g + b
            s_ahead = (_LOOK + b + _LOOK) % _NBUF
            wait_write(s_ahead)
            start_gather(j + _LOOK, s_ahead)
            slot = (_LOOK + b) % _NBUF
            wait_gather(slot)
            start_write(j, slot)

    for t in range(_LOOK):
        slot = (_N - _LOOK + t) % _NBUF
        wait_gather(slot)
        start_write(_N - _LOOK + t, slot)

    for b in range(_NBUF):
        wait_write(b)


def kernel(inputs, context_weight):
    idx = inputs.reshape(-1).astype(jnp.int32)
    out = _gather_kernel(idx, context_weight)
    return out.reshape(inputs.shape[0], inputs.shape[1], _EMBD)


# speedup vs baseline: 1.7632x; 1.5865x over previous
import functools

import jax
import jax.numpy as jnp
from jax import lax
from jax.experimental import pallas as pl
from jax.experimental.pallas import tpu as pltpu
from jax.experimental.pallas import tpu_sc as plsc

_EMBD = 32
_BATCH = 16384
_HIST = 50

_NC = 2
_NS = 16
_NW = _NC * _NS
_BPW = _BATCH // _NW
_NBUF = 8
_LOOK = 4

_mesh = plsc.VectorSubcoreMesh(core_axis_name="c", subcore_axis_name="s")


@functools.partial(
    pl.kernel,
    mesh=_mesh,
    out_type=jax.ShapeDtypeStruct((_BATCH, _HIST, _EMBD), jnp.float32),
    scratch_types=(
        [pltpu.VMEM((_BPW, _HIST), jnp.int32)]
        + [pltpu.VMEM((_HIST, _EMBD), jnp.float32) for _ in range(_NBUF)]
        + [pltpu.SemaphoreType.DMA for _ in range(2 * _NBUF)]
    ),
    compiler_params=pltpu.CompilerParams(use_tc_tiling_on_sc=False),
)
def _gather_kernel(idx_hbm, table_hbm, out_hbm, idx_v, *bufs_and_sems):
    bufs = bufs_and_sems[:_NBUF]
    gsem = bufs_and_sems[_NBUF : 2 * _NBUF]
    wsem = bufs_and_sems[2 * _NBUF :]

    wid = lax.axis_index("s") * _NC + lax.axis_index("c")
    base = wid * _BPW
    pltpu.sync_copy(idx_hbm.at[pl.ds(base, _BPW)], idx_v)

    def start_gather(j, slot):
        pltpu.async_copy(table_hbm.at[idx_v.at[j]], bufs[slot], gsem[slot])

    def wait_gather(slot):
        pltpu.make_async_copy(
            table_hbm.at[idx_v.at[0]], bufs[slot], gsem[slot]
        ).wait()

    def start_write(j, slot):
        pltpu.async_copy(bufs[slot], out_hbm.at[base + j], wsem[slot])

    def wait_write(slot):
        pltpu.make_async_copy(bufs[slot], out_hbm.at[0], wsem[slot]).wait()

    for j in range(_LOOK):
        start_gather(j, j)

    for j in range(_LOOK):
        start_gather(j + _LOOK, j + _LOOK)
        wait_gather(j)
        start_write(j, j)

    @pl.loop(_LOOK, _BPW - _LOOK, step=_NBUF)
    def _steady(g):
        for b in range(_NBUF):
            j = g + b
            s_ahead = (_LOOK + b + _LOOK) % _NBUF
            wait_write(s_ahead)
            start_gather(j + _LOOK, s_ahead)
            slot = (_LOOK + b) % _NBUF
            wait_gather(slot)
            start_write(j, slot)

    for t in range(_LOOK):
        slot = (_BPW - _LOOK + t) % _NBUF
        wait_gather(slot)
        start_write(_BPW - _LOOK + t, slot)

    for b in range(_NBUF):
        wait_write(b)


def kernel(inputs, context_weight):
    return _gather_kernel(inputs.astype(jnp.int32), context_weight)
